# baseline (speedup 1.0000x reference)
import functools
import math

import numpy as np
import jax
import jax.numpy as jnp
from jax import lax
from jax.experimental import pallas as pl
from jax.experimental.pallas import tpu as pltpu
from jax.experimental.pallas import tpu_sc as plsc

NUM_SPECIES = 100
EMBED_DIM = 8
NUM_BASIS = 16
CUTOFF = 5.0
N_NODES = 100000
N_EDGES = 1600000


_NC, _NS = 2, 16
_NW = _NC * _NS
_PER_W = 3200
_N_PAD = _NW * _PER_W
_WIDTH = 2 * EMBED_DIM


def _node_gather_body(x_hbm, w_hbm, outx_hbm, outz_hbm, idx_v, tab_v, rxt_v, rzt_v, sem):
    wid = lax.axis_index("s") * _NC + lax.axis_index("c")
    base = wid * _PER_W
    h_idx = pltpu.async_copy(x_hbm.at[pl.ds(base, _PER_W)], idx_v, sem)
    pltpu.sync_copy(w_hbm, tab_v)
    h_idx.wait()

    def group(g, _):
        idx16 = idx_v[pl.ds(g * 16, 16)]
        fbase = idx16 * _WIDTH
        for j in range(_WIDTH):
            vals = plsc.load_gather(tab_v, [fbase + j])
            buf = rxt_v if j < EMBED_DIM else rzt_v
            buf[j % EMBED_DIM, pl.ds(g * 16, 16)] = vals
        return 0

    lax.fori_loop(0, _PER_W // 16, group, 0)
    handles = []
    for j in range(EMBED_DIM):
        handles.append(pltpu.async_copy(rxt_v.at[j], outx_hbm.at[j, pl.ds(base, _PER_W)], sem))
        handles.append(pltpu.async_copy(rzt_v.at[j], outz_hbm.at[j, pl.ds(base, _PER_W)], sem))
    for h in handles:
        h.wait()


@functools.cache
def _node_gather():
    return pl.kernel(
        _node_gather_body,
        mesh=plsc.VectorSubcoreMesh(core_axis_name="c", subcore_axis_name="s"),
        compiler_params=pltpu.CompilerParams(needs_layout_passes=False),
        out_type=[
            jax.ShapeDtypeStruct((EMBED_DIM, _N_PAD), jnp.float32),
            jax.ShapeDtypeStruct((EMBED_DIM, _N_PAD), jnp.float32),
        ],
        scratch_types=[
            pltpu.VMEM((_PER_W,), jnp.int32),
            pltpu.VMEM((NUM_SPECIES * _WIDTH,), jnp.float32),
            pltpu.VMEM((EMBED_DIM, _PER_W), jnp.float32),
            pltpu.VMEM((EMBED_DIM, _PER_W), jnp.float32),
            pltpu.SemaphoreType.DMA,
        ],
    )


_NODE_BLK = 2048
_NODE_GRID = -(-N_NODES // _NODE_BLK)


def _node_t_body(xt_ref, zt_ref, ox_ref, oz_ref):
    ox_ref[...] = jnp.transpose(xt_ref[...])
    oz_ref[...] = jnp.transpose(zt_ref[...])


def _node_transpose(fxt, fzt):
    return pl.pallas_call(
        _node_t_body,
        grid=(_NODE_GRID,),
        in_specs=[
            pl.BlockSpec((EMBED_DIM, _NODE_BLK), lambda i: (0, i)),
            pl.BlockSpec((EMBED_DIM, _NODE_BLK), lambda i: (0, i)),
        ],
        out_specs=[
            pl.BlockSpec((_NODE_BLK, EMBED_DIM), lambda i: (i, 0)),
            pl.BlockSpec((_NODE_BLK, EMBED_DIM), lambda i: (i, 0)),
        ],
        out_shape=[
            jax.ShapeDtypeStruct((N_NODES, EMBED_DIM), jnp.float32),
            jax.ShapeDtypeStruct((N_NODES, EMBED_DIM), jnp.float32),
        ],
    )(fxt, fzt)


_EDGE_BLK = 6400

_NSUB = 4
_SUB = _EDGE_BLK // _NSUB


def _edge_body(e_hbm, c_ref, o_hbm, ei_ref, eo_ref, s_refs, sin_ref, sout_ref):
    i = pl.program_id(0)
    ng = pl.num_programs(0)
    par = lax.rem(i, 2)
    nxt = 1 - par

    def in_copy(step, k, buf):
        base = step * _EDGE_BLK + k * _SUB
        return pltpu.make_async_copy(
            e_hbm.at[pl.ds(base, _SUB)],
            ei_ref.at[buf, k],
            sin_ref.at[buf, k],
        )

    @pl.when(i == 0)
    def _():
        for k in range(_NSUB):
            in_copy(0, k, 0).start()

    @pl.when(i + 1 < ng)
    def _():
        for k in range(_NSUB):
            in_copy(i + 1, k, nxt).start()

    eye3 = (lax.broadcasted_iota(jnp.int32, (3, 3), 0)
            == lax.broadcasted_iota(jnp.int32, (3, 3), 1)).astype(jnp.float32)
    for k in range(_NSUB):
        s_ref = s_refs.at[k]
        in_copy(i, k, par).wait()
        e = ei_ref[par, k]
        t3 = lax.dot_general(eye3, e, (((1,), (1,)), ((), ())),
                             preferred_element_type=jnp.float32)
        xr = t3[0:1, :]
        yr = t3[1:2, :]
        zr = t3[2:3, :]
        r2 = xr * xr + yr * yr + zr * zr
        r = jnp.sqrt(r2)
        theta = r * (math.pi / CUTOFF)
        q = jnp.round(theta * (2.0 / math.pi))
        t = theta - q * (math.pi / 2.0)
        t2 = t * t
        st = t * (1.0 + t2 * (-1.0 / 6.0 + t2 * (1.0 / 120.0 + t2 * (-1.0 / 5040.0))))
        ct = 1.0 + t2 * (-0.5 + t2 * (1.0 / 24.0 + t2 * (-1.0 / 720.0 + t2 * (1.0 / 40320.0))))
        qm = jnp.bitwise_and(q.astype(jnp.int32), 3)
        bit0 = jnp.bitwise_and(qm, 1) == 1
        sin_sign = jnp.where(qm >= 2, -1.0, 1.0)
        cos_sign = jnp.where(jnp.logical_or(qm == 1, qm == 2), -1.0, 1.0)
        sin1 = sin_sign * jnp.where(bit0, ct, st)
        cos1 = cos_sign * jnp.where(bit0, st, ct)
        s1 = (math.sqrt(2.0 / CUTOFF) / r) * sin1
        c2x = 2.0 * cos1
        s_pp = jnp.zeros_like(s1)
        s_p = s1
        s_ref[pl.ds(0, 1), :] = s1
        for n in range(1, NUM_BASIS):
            s_n = c2x * s_p - s_pp
            s_ref[pl.ds(n, 1), :] = s_n
            s_pp, s_p = s_p, s_n
        res = lax.dot_general(
            s_ref[...], c_ref[...], (((0,), (0,)), ((), ())),
            preferred_element_type=jnp.float32)

        @pl.when(i > 0)
        def _():
            pltpu.make_async_copy(eo_ref.at[k], o_hbm.at[pl.ds(0, _SUB)],
                                  sout_ref.at[k]).wait()

        eo_ref[k] = res
        base = i * _EDGE_BLK + k * _SUB
        pltpu.make_async_copy(eo_ref.at[k], o_hbm.at[pl.ds(base, _SUB)],
                              sout_ref.at[k]).start()

    @pl.when(i == ng - 1)
    def _():
        for k in range(_NSUB):
            pltpu.make_async_copy(eo_ref.at[k], o_hbm.at[pl.ds(0, _SUB)],
                                  sout_ref.at[k]).wait()


def _edge_call(edge_attr):
    grid = N_EDGES // _EDGE_BLK
    return pl.pallas_call(
        _edge_body,
        grid=(grid,),
        in_specs=[
            pl.BlockSpec(memory_space=pltpu.MemorySpace.HBM),
            pl.BlockSpec((NUM_BASIS, NUM_BASIS), lambda i: (0, 0)),
        ],
        out_specs=pl.BlockSpec(memory_space=pltpu.MemorySpace.HBM),
        out_shape=jax.ShapeDtypeStruct((N_EDGES, NUM_BASIS), jnp.float32),
        scratch_shapes=[
            pltpu.VMEM((2, _NSUB, _SUB, 3), jnp.float32),
            pltpu.VMEM((_NSUB, _SUB, NUM_BASIS), jnp.float32),
            pltpu.VMEM((_NSUB, NUM_BASIS, _SUB), jnp.float32),
            pltpu.SemaphoreType.DMA((2, _NSUB)),
            pltpu.SemaphoreType.DMA((_NSUB,)),
        ],
    )(edge_attr, jnp.eye(NUM_BASIS, dtype=jnp.float32))


def kernel(x, edge_attr, W_x, W_z):
    w_flat = jnp.concatenate([W_x, W_z], axis=1).reshape(-1)
    x_pad = jnp.pad(x.astype(jnp.int32), (0, _N_PAD - N_NODES))
    fxt, fzt = _node_gather()(x_pad, w_flat)
    h_node_x, h_node_z = _node_transpose(fxt, fzt)
    h_edge = _edge_call(edge_attr)
    return (h_node_x, h_node_z, h_edge)


# speedup vs baseline: 5.0638x; 5.0638x over previous
import functools
import math

import jax
import jax.numpy as jnp
from jax import lax
from jax.experimental import pallas as pl
from jax.experimental.pallas import tpu as pltpu
from jax.experimental.pallas import tpu_sc as plsc

NUM_SPECIES = 100
EMBED_DIM = 8
NUM_BASIS = 16
CUTOFF = 5.0
N_NODES = 100000
N_EDGES = 1600000


_NC, _NS = 2, 16
_NW = _NC * _NS
_PER_W = 3200
_WIDTH = 2 * EMBED_DIM


def _node_gather_body(x_hbm, w_hbm, outx_hbm, outz_hbm, idx_v, tab_v, rx_v, rz_v, sem):
    wid = lax.axis_index("s") * _NC + lax.axis_index("c")
    base = jnp.minimum(wid * _PER_W, N_NODES - _PER_W)
    h_idx = pltpu.async_copy(x_hbm.at[pl.ds(base, _PER_W)], idx_v, sem)
    pltpu.sync_copy(w_hbm, tab_v)
    h_idx.wait()
    lanes = lax.iota(jnp.int32, 16)

    def group(g, _):
        idx16 = idx_v[pl.ds(g * 16, 16)]
        fbase = idx16 * _WIDTH
        pos = g * (16 * EMBED_DIM) + lanes * EMBED_DIM
        for j in range(_WIDTH):
            vals = plsc.load_gather(tab_v, [fbase + j])
            buf = rx_v if j < EMBED_DIM else rz_v
            plsc.store_scatter(buf, [pos + (j % EMBED_DIM)], vals)
        return 0

    lax.fori_loop(0, _PER_W // 16, group, 0)
    fl = _PER_W * EMBED_DIM
    h1 = pltpu.async_copy(rx_v, outx_hbm.at[pl.ds(base * EMBED_DIM, fl)], sem)
    h2 = pltpu.async_copy(rz_v, outz_hbm.at[pl.ds(base * EMBED_DIM, fl)], sem)
    h1.wait()
    h2.wait()


@functools.cache
def _node_gather():
    fl = _PER_W * EMBED_DIM
    return pl.kernel(
        _node_gather_body,
        mesh=plsc.VectorSubcoreMesh(core_axis_name="c", subcore_axis_name="s"),
        compiler_params=pltpu.CompilerParams(needs_layout_passes=False),
        out_type=[
            jax.ShapeDtypeStruct((N_NODES * EMBED_DIM,), jnp.float32),
            jax.ShapeDtypeStruct((N_NODES * EMBED_DIM,), jnp.float32),
        ],
        scratch_types=[
            pltpu.VMEM((_PER_W,), jnp.int32),
            pltpu.VMEM((NUM_SPECIES * _WIDTH,), jnp.float32),
            pltpu.VMEM((fl,), jnp.float32),
            pltpu.VMEM((fl,), jnp.float32),
            pltpu.SemaphoreType.DMA,
        ],
    )


_EDGE_BLK = 6400


def _edge_body(e_ref, c_ref, o_ref):
    e = e_ref[...]
    e2 = e * e
    ones = jnp.ones((1, 3), jnp.float32)
    r2 = lax.dot_general(ones, e2, (((1,), (1,)), ((), ())),
                         preferred_element_type=jnp.float32)
    r = jnp.sqrt(r2)
    theta = r * (math.pi / CUTOFF)
    q = jnp.round(theta * (2.0 / math.pi))
    t = theta - q * (math.pi / 2.0)
    t2 = t * t
    st = t * (1.0 + t2 * (-1.0 / 6.0 + t2 * (1.0 / 120.0 + t2 * (-1.0 / 5040.0))))
    ct = 1.0 + t2 * (-0.5 + t2 * (1.0 / 24.0 + t2 * (-1.0 / 720.0 + t2 * (1.0 / 40320.0))))
    qm = jnp.bitwise_and(q.astype(jnp.int32), 3)
    bit0 = jnp.bitwise_and(qm, 1) == 1
    sin_sign = jnp.where(qm >= 2, -1.0, 1.0)
    cos_sign = jnp.where(jnp.logical_or(qm == 1, qm == 2), -1.0, 1.0)
    sin1 = sin_sign * jnp.where(bit0, ct, st)
    cos1 = cos_sign * jnp.where(bit0, st, ct)
    s1 = (math.sqrt(2.0 / CUTOFF) / r) * sin1
    c2x = 2.0 * cos1
    s_pp = jnp.zeros_like(s1)
    s_p = s1
    rows = [s1]
    for _ in range(7):
        s_n = c2x * s_p - s_pp
        rows.append(s_n)
        s_pp, s_p = s_p, s_n
    cos2t = cos1 * cos1 * 2.0 - 1.0
    cos4t = cos2t * cos2t * 2.0 - 1.0
    cos8t = cos4t * cos4t * 2.0 - 1.0
    low = jnp.concatenate(rows, axis=0)
    neg = jnp.concatenate(list(reversed(rows[:7])) + [jnp.zeros_like(s1)],
                          axis=0)
    high = (2.0 * cos8t) * low + neg
    s_all = jnp.concatenate([low, high], axis=0)
    o_ref[...] = lax.dot_general(s_all, c_ref[...], (((0,), (0,)), ((), ())),
                                 preferred_element_type=jnp.float32)


def _edge_call(edge_attr):
    grid = N_EDGES // _EDGE_BLK
    return pl.pallas_call(
        _edge_body,
        grid=(grid,),
        in_specs=[
            pl.BlockSpec((_EDGE_BLK, 3), lambda i: (i, 0)),
            pl.BlockSpec((NUM_BASIS, NUM_BASIS), lambda i: (0, 0)),
        ],
        out_specs=pl.BlockSpec((_EDGE_BLK, NUM_BASIS), lambda i: (i, 0)),
        out_shape=jax.ShapeDtypeStruct((N_EDGES, NUM_BASIS), jnp.float32),
    )(edge_attr, jnp.eye(NUM_BASIS, dtype=jnp.float32))


def kernel(x, edge_attr, W_x, W_z):
    w_flat = jnp.concatenate([W_x, W_z], axis=1).reshape(-1)
    fx, fz = _node_gather()(x.astype(jnp.int32), w_flat)
    h_node_x = fx.reshape(N_NODES, EMBED_DIM)
    h_node_z = fz.reshape(N_NODES, EMBED_DIM)
    h_edge = _edge_call(edge_attr)
    return (h_node_x, h_node_z, h_edge)


# speedup vs baseline: 6.1557x; 1.2156x over previous
import functools
import math

import jax
import jax.numpy as jnp
from jax import lax
from jax.experimental import pallas as pl
from jax.experimental.pallas import tpu as pltpu
from jax.experimental.pallas import tpu_sc as plsc

NUM_SPECIES = 100
EMBED_DIM = 8
NUM_BASIS = 16
CUTOFF = 5.0
N_NODES = 100000
N_EDGES = 1600000


_NC, _NS = 2, 16
_NW = _NC * _NS
_PER_W = 3200
_WIDTH = 2 * EMBED_DIM


def _node_gather_body(x_hbm, w_hbm, outx_hbm, outz_hbm, idx_v, tab_v, rx_v, rz_v, sem):
    wid = lax.axis_index("s") * _NC + lax.axis_index("c")
    base = jnp.minimum(wid * _PER_W, N_NODES - _PER_W)
    h_idx = pltpu.async_copy(x_hbm.at[pl.ds(base, _PER_W)], idx_v, sem)
    pltpu.sync_copy(w_hbm, tab_v)
    h_idx.wait()
    lanes = lax.iota(jnp.int32, 16)

    def group(g, _):
        idx16 = idx_v[pl.ds(g * 16, 16)]
        fbase = idx16 * _WIDTH
        pos = g * (16 * EMBED_DIM) + lanes * EMBED_DIM
        for j in range(_WIDTH):
            vals = plsc.load_gather(tab_v, [fbase + j])
            buf = rx_v if j < EMBED_DIM else rz_v
            plsc.store_scatter(buf, [pos + (j % EMBED_DIM)], vals)
        return 0

    lax.fori_loop(0, _PER_W // 16, group, 0)
    fl = _PER_W * EMBED_DIM
    h1 = pltpu.async_copy(rx_v, outx_hbm.at[pl.ds(base * EMBED_DIM, fl)], sem)
    h2 = pltpu.async_copy(rz_v, outz_hbm.at[pl.ds(base * EMBED_DIM, fl)], sem)
    h1.wait()
    h2.wait()


@functools.cache
def _node_gather():
    fl = _PER_W * EMBED_DIM
    return pl.kernel(
        _node_gather_body,
        mesh=plsc.VectorSubcoreMesh(core_axis_name="c", subcore_axis_name="s"),
        compiler_params=pltpu.CompilerParams(needs_layout_passes=False),
        out_type=[
            jax.ShapeDtypeStruct((N_NODES * EMBED_DIM,), jnp.float32),
            jax.ShapeDtypeStruct((N_NODES * EMBED_DIM,), jnp.float32),
        ],
        scratch_types=[
            pltpu.VMEM((_PER_W,), jnp.int32),
            pltpu.VMEM((NUM_SPECIES * _WIDTH,), jnp.float32),
            pltpu.VMEM((fl,), jnp.float32),
            pltpu.VMEM((fl,), jnp.float32),
            pltpu.SemaphoreType.DMA,
        ],
    )


_EDGE_BLK = 6400


def _edge_body(e_ref, o_ref):
    e = e_ref[...]
    xr = e[0:1, :]
    yr = e[1:2, :]
    zr = e[2:3, :]
    r2 = xr * xr + yr * yr + zr * zr
    r = jnp.sqrt(r2)
    theta = r * (math.pi / CUTOFF)
    q = jnp.round(theta * (2.0 / math.pi))
    t = theta - q * (math.pi / 2.0)
    t2 = t * t
    st = t * (1.0 + t2 * (-1.0 / 6.0 + t2 * (1.0 / 120.0 + t2 * (-1.0 / 5040.0))))
    ct = 1.0 + t2 * (-0.5 + t2 * (1.0 / 24.0 + t2 * (-1.0 / 720.0 + t2 * (1.0 / 40320.0))))
    qm = jnp.bitwise_and(q.astype(jnp.int32), 3)
    bit0 = jnp.bitwise_and(qm, 1) == 1
    sin_sign = jnp.where(qm >= 2, -1.0, 1.0)
    cos_sign = jnp.where(jnp.logical_or(qm == 1, qm == 2), -1.0, 1.0)
    sin1 = sin_sign * jnp.where(bit0, ct, st)
    cos1 = cos_sign * jnp.where(bit0, st, ct)
    s1 = (math.sqrt(2.0 / CUTOFF) / r) * sin1
    c2x = 2.0 * cos1
    s_pp = jnp.zeros_like(s1)
    s_p = s1
    o_ref[pl.ds(0, 1), :] = s1
    for n in range(1, NUM_BASIS):
        s_n = c2x * s_p - s_pp
        o_ref[pl.ds(n, 1), :] = s_n
        s_pp, s_p = s_p, s_n


def _edge_call(edge_attr_t):
    grid = N_EDGES // _EDGE_BLK
    return pl.pallas_call(
        _edge_body,
        grid=(grid,),
        in_specs=[pl.BlockSpec((3, _EDGE_BLK), lambda i: (0, i))],
        out_specs=pl.BlockSpec((NUM_BASIS, _EDGE_BLK), lambda i: (0, i)),
        out_shape=jax.ShapeDtypeStruct((NUM_BASIS, N_EDGES), jnp.float32),
    )(edge_attr_t)


def kernel(x, edge_attr, W_x, W_z):
    w_flat = jnp.concatenate([W_x, W_z], axis=1).reshape(-1)
    fx, fz = _node_gather()(x.astype(jnp.int32), w_flat)
    h_node_x = fx.reshape(N_NODES, EMBED_DIM)
    h_node_z = fz.reshape(N_NODES, EMBED_DIM)
    h_edge = jnp.transpose(_edge_call(jnp.transpose(edge_attr)))
    return (h_node_x, h_node_z, h_edge)


# speedup vs baseline: 11.1292x; 1.8080x over previous
import functools
import math

import jax
import jax.numpy as jnp
from jax import lax
from jax.experimental import pallas as pl
from jax.experimental.pallas import tpu as pltpu
from jax.experimental.pallas import tpu_sc as plsc

NUM_SPECIES = 100
EMBED_DIM = 8
NUM_BASIS = 16
CUTOFF = 5.0
N_NODES = 100000
N_EDGES = 1600000


_NC, _NS = 2, 16
_NW = _NC * _NS
_PER_W = 3200
_N_PAD = _NW * _PER_W
_WIDTH = 2 * EMBED_DIM


def _node_gather_body(x_hbm, w_hbm, outx_hbm, outz_hbm, idx_v, tab_v, rxt_v, rzt_v, sem):
    wid = lax.axis_index("s") * _NC + lax.axis_index("c")
    base = wid * _PER_W
    h_idx = pltpu.async_copy(x_hbm.at[pl.ds(base, _PER_W)], idx_v, sem)
    pltpu.sync_copy(w_hbm, tab_v)
    h_idx.wait()

    def group(g, _):
        idx16 = idx_v[pl.ds(g * 16, 16)]
        fbase = idx16 * _WIDTH
        for j in range(_WIDTH):
            vals = plsc.load_gather(tab_v, [fbase + j])
            buf = rxt_v if j < EMBED_DIM else rzt_v
            buf[j % EMBED_DIM, pl.ds(g * 16, 16)] = vals
        return 0

    lax.fori_loop(0, _PER_W // 16, group, 0)
    handles = []
    for j in range(EMBED_DIM):
        handles.append(pltpu.async_copy(rxt_v.at[j], outx_hbm.at[j, pl.ds(base, _PER_W)], sem))
        handles.append(pltpu.async_copy(rzt_v.at[j], outz_hbm.at[j, pl.ds(base, _PER_W)], sem))
    for h in handles:
        h.wait()


@functools.cache
def _node_gather():
    return pl.kernel(
        _node_gather_body,
        mesh=plsc.VectorSubcoreMesh(core_axis_name="c", subcore_axis_name="s"),
        compiler_params=pltpu.CompilerParams(needs_layout_passes=False),
        out_type=[
            jax.ShapeDtypeStruct((EMBED_DIM, _N_PAD), jnp.float32),
            jax.ShapeDtypeStruct((EMBED_DIM, _N_PAD), jnp.float32),
        ],
        scratch_types=[
            pltpu.VMEM((_PER_W,), jnp.int32),
            pltpu.VMEM((NUM_SPECIES * _WIDTH,), jnp.float32),
            pltpu.VMEM((EMBED_DIM, _PER_W), jnp.float32),
            pltpu.VMEM((EMBED_DIM, _PER_W), jnp.float32),
            pltpu.SemaphoreType.DMA,
        ],
    )


_EDGE_BLK = 12800


def _edge_body(e_ref, o_ref):
    e = e_ref[...]
    xr = e[0:1, :]
    yr = e[1:2, :]
    zr = e[2:3, :]
    r2 = xr * xr + yr * yr + zr * zr
    r = jnp.sqrt(r2)
    theta = r * (math.pi / CUTOFF)
    q = jnp.round(theta * (2.0 / math.pi))
    t = theta - q * (math.pi / 2.0)
    t2 = t * t
    st = t * (1.0 + t2 * (-1.0 / 6.0 + t2 * (1.0 / 120.0 + t2 * (-1.0 / 5040.0))))
    ct = 1.0 + t2 * (-0.5 + t2 * (1.0 / 24.0 + t2 * (-1.0 / 720.0 + t2 * (1.0 / 40320.0))))
    qm = jnp.bitwise_and(q.astype(jnp.int32), 3)
    bit0 = jnp.bitwise_and(qm, 1) == 1
    sin_sign = jnp.where(qm >= 2, -1.0, 1.0)
    cos_sign = jnp.where(jnp.logical_or(qm == 1, qm == 2), -1.0, 1.0)
    sin1 = sin_sign * jnp.where(bit0, ct, st)
    cos1 = cos_sign * jnp.where(bit0, st, ct)
    s1 = (math.sqrt(2.0 / CUTOFF) / r) * sin1
    c2x = 2.0 * cos1
    s_pp = jnp.zeros_like(s1)
    s_p = s1
    o_ref[pl.ds(0, 1), :] = s1
    for n in range(1, NUM_BASIS):
        s_n = c2x * s_p - s_pp
        o_ref[pl.ds(n, 1), :] = s_n
        s_pp, s_p = s_p, s_n


def _edge_call(edge_attr_t):
    grid = N_EDGES // _EDGE_BLK
    return pl.pallas_call(
        _edge_body,
        grid=(grid,),
        in_specs=[pl.BlockSpec((3, _EDGE_BLK), lambda i: (0, i))],
        out_specs=pl.BlockSpec((NUM_BASIS, _EDGE_BLK), lambda i: (0, i)),
        out_shape=jax.ShapeDtypeStruct((NUM_BASIS, N_EDGES), jnp.float32),
    )(edge_attr_t)


def kernel(x, edge_attr, W_x, W_z):
    w_flat = jnp.concatenate([W_x, W_z], axis=1).reshape(-1)
    x_pad = jnp.pad(x.astype(jnp.int32), (0, _N_PAD - N_NODES))
    fxt, fzt = _node_gather()(x_pad, w_flat)
    h_node_x = jnp.transpose(fxt)[:N_NODES]
    h_node_z = jnp.transpose(fzt)[:N_NODES]
    h_edge = jnp.transpose(_edge_call(jnp.transpose(edge_attr)))
    return (h_node_x, h_node_z, h_edge)
